# baseline (speedup 1.0000x reference)
import jax
import jax.numpy as jnp
from jax.experimental import pallas as pl
from jax.experimental.pallas import tpu as pltpu


def _aug_lhs(xb):
    nb = (xb[:, 0:1] * xb[:, 0:1] + xb[:, 1:2] * xb[:, 1:2]
          + xb[:, 2:3] * xb[:, 2:3])
    ones = jnp.ones_like(nb)
    l = jnp.concatenate([xb * (-2.0), nb, ones], axis=1)
    lhi = l.astype(jnp.bfloat16)
    llo = (l - lhi.astype(jnp.float32)).astype(jnp.bfloat16)
    return jnp.concatenate([lhi, lhi, llo], axis=1)


def _aug_rhs(t):
    n = t[0:1, :] * t[0:1, :] + t[1:2, :] * t[1:2, :] + t[2:3, :] * t[2:3, :]
    ones = jnp.ones_like(n)
    r = jnp.concatenate([t, ones, n], axis=0)
    rhi = r.astype(jnp.bfloat16)
    rlo = (r - rhi.astype(jnp.float32)).astype(jnp.bfloat16)
    return jnp.concatenate([rhi, rlo, rhi], axis=0)


def _chamfer_tc_kernel(x1_ref, x2_ref, x1t_ref, x2t_ref, d1_ref, d2_ref):
    dims = (((1,), (0,)), ((), ()))
    da = jax.lax.dot_general(_aug_lhs(x1_ref[0]), _aug_rhs(x2t_ref[0]),
                             dimension_numbers=dims,
                             preferred_element_type=jnp.float32)
    d2_ref[0, 0, :] = jnp.min(da, axis=0)
    db = jax.lax.dot_general(_aug_lhs(x2_ref[0]), _aug_rhs(x1t_ref[0]),
                             dimension_numbers=dims,
                             preferred_element_type=jnp.float32)
    d1_ref[0, 0, :] = jnp.min(db, axis=0)


def kernel(xyz1, xyz2):
    B, N, _ = xyz1.shape
    M = xyz2.shape[1]
    x1t = jnp.swapaxes(xyz1, 1, 2)
    x2t = jnp.swapaxes(xyz2, 1, 2)
    d1, d2 = pl.pallas_call(
        _chamfer_tc_kernel,
        grid=(B,),
        in_specs=[
            pl.BlockSpec((1, N, 3), lambda b: (b, 0, 0)),
            pl.BlockSpec((1, M, 3), lambda b: (b, 0, 0)),
            pl.BlockSpec((1, 3, N), lambda b: (b, 0, 0)),
            pl.BlockSpec((1, 3, M), lambda b: (b, 0, 0)),
        ],
        out_specs=[
            pl.BlockSpec((1, 1, N), lambda b: (b, 0, 0)),
            pl.BlockSpec((1, 1, M), lambda b: (b, 0, 0)),
        ],
        out_shape=[
            jax.ShapeDtypeStruct((B, 1, N), jnp.float32),
            jax.ShapeDtypeStruct((B, 1, M), jnp.float32),
        ],
        compiler_params=pltpu.CompilerParams(
            dimension_semantics=("parallel",)),
    )(xyz1, xyz2, x1t, x2t)
    return d1.reshape(B, N), d2.reshape(B, M)


# speedup vs baseline: 1.4040x; 1.4040x over previous
import jax
import jax.numpy as jnp
from jax.experimental import pallas as pl
from jax.experimental.pallas import tpu as pltpu


def _aug_query(t):
    n = t[0:1, :] * t[0:1, :] + t[1:2, :] * t[1:2, :] + t[2:3, :] * t[2:3, :]
    ones = jnp.ones_like(n)
    l = jnp.concatenate([t * (-2.0), n, ones], axis=0)
    lhi = l.astype(jnp.bfloat16)
    llo = (l - lhi.astype(jnp.float32)).astype(jnp.bfloat16)
    return jnp.concatenate([lhi, lhi, llo], axis=0)


def _aug_key(t):
    n = t[0:1, :] * t[0:1, :] + t[1:2, :] * t[1:2, :] + t[2:3, :] * t[2:3, :]
    ones = jnp.ones_like(n)
    r = jnp.concatenate([t, ones, n], axis=0)
    rhi = r.astype(jnp.bfloat16)
    rlo = (r - rhi.astype(jnp.float32)).astype(jnp.bfloat16)
    return jnp.concatenate([rhi, rlo, rhi], axis=0)


def _chamfer_tc_kernel(x1t_ref, x2t_ref, d1_ref, d2_ref):
    x1t = x1t_ref[0]
    x2t = x2t_ref[0]
    dims = (((0,), (0,)), ((), ()))
    da = jax.lax.dot_general(_aug_query(x1t), _aug_key(x2t),
                             dimension_numbers=dims,
                             preferred_element_type=jnp.float32)
    d2_ref[0, 0, :] = jnp.min(da, axis=0)
    db = jax.lax.dot_general(_aug_query(x2t), _aug_key(x1t),
                             dimension_numbers=dims,
                             preferred_element_type=jnp.float32)
    d1_ref[0, 0, :] = jnp.min(db, axis=0)


def kernel(xyz1, xyz2):
    B, N, _ = xyz1.shape
    M = xyz2.shape[1]
    x1t = jnp.swapaxes(xyz1, 1, 2)
    x2t = jnp.swapaxes(xyz2, 1, 2)
    d1, d2 = pl.pallas_call(
        _chamfer_tc_kernel,
        grid=(B,),
        in_specs=[
            pl.BlockSpec((1, 3, N), lambda b: (b, 0, 0)),
            pl.BlockSpec((1, 3, M), lambda b: (b, 0, 0)),
        ],
        out_specs=[
            pl.BlockSpec((1, 1, N), lambda b: (b, 0, 0)),
            pl.BlockSpec((1, 1, M), lambda b: (b, 0, 0)),
        ],
        out_shape=[
            jax.ShapeDtypeStruct((B, 1, N), jnp.float32),
            jax.ShapeDtypeStruct((B, 1, M), jnp.float32),
        ],
        compiler_params=pltpu.CompilerParams(
            dimension_semantics=("parallel",)),
    )(x1t, x2t)
    return d1.reshape(B, N), d2.reshape(B, M)


# speedup vs baseline: 1.4353x; 1.0223x over previous
import jax
import jax.numpy as jnp
from jax.experimental import pallas as pl
from jax.experimental.pallas import tpu as pltpu


def _aug_query(t):
    n = t[0:1, :] * t[0:1, :] + t[1:2, :] * t[1:2, :] + t[2:3, :] * t[2:3, :]
    ones = jnp.ones_like(n)
    l = jnp.concatenate([t * (-2.0), n, ones], axis=0)
    lhi = l.astype(jnp.bfloat16)
    llo = (l - lhi.astype(jnp.float32)).astype(jnp.bfloat16)
    return jnp.concatenate([lhi, lhi, llo], axis=0)


def _aug_key(t):
    n = t[0:1, :] * t[0:1, :] + t[1:2, :] * t[1:2, :] + t[2:3, :] * t[2:3, :]
    ones = jnp.ones_like(n)
    r = jnp.concatenate([t, ones, n], axis=0)
    rhi = r.astype(jnp.bfloat16)
    rlo = (r - rhi.astype(jnp.float32)).astype(jnp.bfloat16)
    return jnp.concatenate([rhi, rlo, rhi], axis=0)


_BB = 2


def _chamfer_tc_kernel(x1t_ref, x2t_ref, d1_ref, d2_ref):
    dims = (((0,), (0,)), ((), ()))
    for k in range(_BB):
        x1t = x1t_ref[k]
        x2t = x2t_ref[k]
        da = jax.lax.dot_general(_aug_query(x1t), _aug_key(x2t),
                                 dimension_numbers=dims,
                                 preferred_element_type=jnp.float32)
        d2_ref[k, 0, :] = jnp.min(da, axis=0)
        db = jax.lax.dot_general(_aug_query(x2t), _aug_key(x1t),
                                 dimension_numbers=dims,
                                 preferred_element_type=jnp.float32)
        d1_ref[k, 0, :] = jnp.min(db, axis=0)


def kernel(xyz1, xyz2):
    B, N, _ = xyz1.shape
    M = xyz2.shape[1]
    x1t = jnp.swapaxes(xyz1, 1, 2)
    x2t = jnp.swapaxes(xyz2, 1, 2)
    d1, d2 = pl.pallas_call(
        _chamfer_tc_kernel,
        grid=(B // _BB,),
        in_specs=[
            pl.BlockSpec((_BB, 3, N), lambda b: (b, 0, 0)),
            pl.BlockSpec((_BB, 3, M), lambda b: (b, 0, 0)),
        ],
        out_specs=[
            pl.BlockSpec((_BB, 1, N), lambda b: (b, 0, 0)),
            pl.BlockSpec((_BB, 1, M), lambda b: (b, 0, 0)),
        ],
        out_shape=[
            jax.ShapeDtypeStruct((B, 1, N), jnp.float32),
            jax.ShapeDtypeStruct((B, 1, M), jnp.float32),
        ],
        compiler_params=pltpu.CompilerParams(
            dimension_semantics=("parallel",)),
    )(x1t, x2t)
    return d1.reshape(B, N), d2.reshape(B, M)


# speedup vs baseline: 1.4419x; 1.0046x over previous
import jax
import jax.numpy as jnp
from jax.experimental import pallas as pl
from jax.experimental.pallas import tpu as pltpu


def _aug_query(t):
    n = t[0:1, :] * t[0:1, :] + t[1:2, :] * t[1:2, :] + t[2:3, :] * t[2:3, :]
    ones = jnp.ones_like(n)
    l = jnp.concatenate([t * (-2.0), n, ones], axis=0)
    lhi = l.astype(jnp.bfloat16)
    llo = (l - lhi.astype(jnp.float32)).astype(jnp.bfloat16)
    return jnp.concatenate([lhi, lhi, llo], axis=0)


def _aug_key(t):
    n = t[0:1, :] * t[0:1, :] + t[1:2, :] * t[1:2, :] + t[2:3, :] * t[2:3, :]
    ones = jnp.ones_like(n)
    r = jnp.concatenate([t, ones, n], axis=0)
    rhi = r.astype(jnp.bfloat16)
    rlo = (r - rhi.astype(jnp.float32)).astype(jnp.bfloat16)
    return jnp.concatenate([rhi, rlo, rhi], axis=0)


_BB = 4


def _chamfer_tc_kernel(x1t_ref, x2t_ref, d1_ref, d2_ref):
    dims = (((0,), (0,)), ((), ()))
    for k in range(_BB):
        x1t = x1t_ref[k]
        x2t = x2t_ref[k]
        da = jax.lax.dot_general(_aug_query(x1t), _aug_key(x2t),
                                 dimension_numbers=dims,
                                 preferred_element_type=jnp.float32)
        d2_ref[k, 0, :] = jnp.min(da, axis=0)
        db = jax.lax.dot_general(_aug_query(x2t), _aug_key(x1t),
                                 dimension_numbers=dims,
                                 preferred_element_type=jnp.float32)
        d1_ref[k, 0, :] = jnp.min(db, axis=0)


def kernel(xyz1, xyz2):
    B, N, _ = xyz1.shape
    M = xyz2.shape[1]
    x1t = jnp.swapaxes(xyz1, 1, 2)
    x2t = jnp.swapaxes(xyz2, 1, 2)
    d1, d2 = pl.pallas_call(
        _chamfer_tc_kernel,
        grid=(B // _BB,),
        in_specs=[
            pl.BlockSpec((_BB, 3, N), lambda b: (b, 0, 0)),
            pl.BlockSpec((_BB, 3, M), lambda b: (b, 0, 0)),
        ],
        out_specs=[
            pl.BlockSpec((_BB, 1, N), lambda b: (b, 0, 0)),
            pl.BlockSpec((_BB, 1, M), lambda b: (b, 0, 0)),
        ],
        out_shape=[
            jax.ShapeDtypeStruct((B, 1, N), jnp.float32),
            jax.ShapeDtypeStruct((B, 1, M), jnp.float32),
        ],
        compiler_params=pltpu.CompilerParams(
            dimension_semantics=("parallel",)),
    )(x1t, x2t)
    return d1.reshape(B, N), d2.reshape(B, M)
